# baseline (speedup 1.0000x reference)
import jax
import jax.numpy as jnp
from jax.experimental import pallas as pl


def _fused_kernel(xa_ref, xv_ref, prob_ref, w_ref, asrc_ref, adst_ref,
                  wprobt_ref, bprob_ref, wattt_ref, batt_ref,
                  xa2_ref, xv2_ref, fp_ref, ap_ref, vp_ref):
    xa = xa_ref[0]
    xv = xv_ref[0]
    x = jnp.concatenate([xa, xv], axis=0)
    w = w_ref[...]
    asrc = asrc_ref[...].reshape(1, -1)
    adst = adst_ref[...].reshape(1, -1)

    h = jnp.dot(x, w, preferred_element_type=jnp.float32)

    s_col = jnp.sum(h * asrc, axis=1, keepdims=True)
    t_row = jax.lax.dot_general(adst, h, (((1,), (1,)), ((), ())),
                                preferred_element_type=jnp.float32)

    maxt = jnp.max(t_row, axis=1, keepdims=True)
    y = s_col + maxt
    m = jnp.maximum(y, 0.2 * y)
    a_c = jnp.exp(y - m)
    c_c = jnp.exp(0.2 * y - m)
    b_r = jnp.exp(t_row - maxt)
    d_r = jnp.exp(0.2 * (t_row - maxt))

    page = prob_ref[0]
    C = page.shape[1] // 2
    nodes = jnp.concatenate([page[:, :C], page[:, C:]], axis=0)
    overlap = jax.lax.dot_general(nodes, nodes,
                                  (((1,), (1,)), ((), ())),
                                  preferred_element_type=jnp.float32)
    p32 = jnp.minimum(jnp.maximum(a_c * b_r, c_c * d_r), overlap)
    rs = jnp.sum(p32, axis=1, keepdims=True)
    p = p32.astype(jnp.bfloat16)

    nact = jnp.sum(nodes.astype(jnp.float32), axis=1, keepdims=True)
    empty = jnp.where(nact > 0.0, 0.0, 1.0)

    rcp = 1.0 / jnp.where(rs > 0.0, rs, 1.0)
    agg = jnp.dot(p, h.astype(jnp.bfloat16),
                  preferred_element_type=jnp.float32)
    out = agg * rcp + x + empty * h
    out = jnp.where(out > 0, out, jnp.exp(jnp.minimum(out, 0.0)) - 1.0)

    T = xa.shape[0]
    xa2_ref[0] = out[:T]
    xv2_ref[0] = out[T:]
    f = jax.nn.sigmoid(
        jax.lax.dot_general(out, wprobt_ref[...], (((1,), (1,)), ((), ())),
                            preferred_element_type=jnp.float32)
        + bprob_ref[...].reshape(1, -1))
    fa = f[:T]
    fv = f[T:]
    fp_ref[0, 0] = fa
    fp_ref[0, 1] = fv

    l = jax.lax.dot_general(out, wattt_ref[...], (((1,), (1,)), ((), ())),
                            preferred_element_type=jnp.float32) \
        + batt_ref[...].reshape(1, -1)
    la = l[:T]
    lv = l[T:]
    ea = jnp.exp(la - jnp.max(la, axis=0, keepdims=True))
    ev = jnp.exp(lv - jnp.max(lv, axis=0, keepdims=True))
    num_a = jnp.sum(ea * fa, axis=0, keepdims=True)
    num_v = jnp.sum(ev * fv, axis=0, keepdims=True)
    den_a = jnp.sum(ea, axis=0, keepdims=True)
    den_v = jnp.sum(ev, axis=0, keepdims=True)
    ap_ref[0] = jnp.clip(num_a / den_a, 0.0, 1.0)
    vp_ref[0] = jnp.clip(num_v / den_v, 0.0, 1.0)


def kernel(x_a, x_v, s1_frame_prob, W, a_src, a_dst, W_prob, b_prob, W_att, b_att):
    bs, T, d = x_a.shape
    C = s1_frame_prob.shape[-1]
    n2 = 2 * T

    prob2 = (s1_frame_prob > 0.5).astype(jnp.bfloat16).reshape(bs, T, 2 * C)
    wprobt = W_prob.T
    wattt = W_att.T

    grid = (bs,)
    xa2, xv2, fp, ap, vp = pl.pallas_call(
        _fused_kernel,
        grid=grid,
        in_specs=[
            pl.BlockSpec((1, T, d), lambda b: (b, 0, 0)),
            pl.BlockSpec((1, T, d), lambda b: (b, 0, 0)),
            pl.BlockSpec((1, T, 2 * C), lambda b: (b, 0, 0)),
            pl.BlockSpec((d, d), lambda b: (0, 0)),
            pl.BlockSpec((d,), lambda b: (0,)),
            pl.BlockSpec((d,), lambda b: (0,)),
            pl.BlockSpec((C, d), lambda b: (0, 0)),
            pl.BlockSpec((C,), lambda b: (0,)),
            pl.BlockSpec((C, d), lambda b: (0, 0)),
            pl.BlockSpec((C,), lambda b: (0,)),
        ],
        out_specs=[
            pl.BlockSpec((1, T, d), lambda b: (b, 0, 0)),
            pl.BlockSpec((1, T, d), lambda b: (b, 0, 0)),
            pl.BlockSpec((1, 2, T, C), lambda b: (b, 0, 0, 0)),
            pl.BlockSpec((1, 1, C), lambda b: (b, 0, 0)),
            pl.BlockSpec((1, 1, C), lambda b: (b, 0, 0)),
        ],
        out_shape=[
            jax.ShapeDtypeStruct((bs, T, d), jnp.float32),
            jax.ShapeDtypeStruct((bs, T, d), jnp.float32),
            jax.ShapeDtypeStruct((bs, 2, T, C), jnp.float32),
            jax.ShapeDtypeStruct((bs, 1, C), jnp.float32),
            jax.ShapeDtypeStruct((bs, 1, C), jnp.float32),
        ],
    )(x_a, x_v, prob2, W, a_src, a_dst, wprobt, b_prob, wattt, b_att)

    frame_prob = jnp.transpose(fp, (0, 2, 1, 3))
    a_prob = ap[:, 0, :]
    v_prob = vp[:, 0, :]
    zeros_event = jnp.zeros((bs, C, d), dtype=jnp.float32)
    return (a_prob, v_prob, frame_prob, xa2, xv2, zeros_event, zeros_event)


# speedup vs baseline: 1.0250x; 1.0250x over previous
import jax
import jax.numpy as jnp
from jax.experimental import pallas as pl


def _fused_kernel(xa_ref, xv_ref, prob_ref, w_ref, asrc_ref, adst_ref,
                  wprobt_ref, bprob_ref, wattt_ref, batt_ref,
                  xa2_ref, xv2_ref, fp_ref, ap_ref, vp_ref):
    xa = xa_ref[0]
    xv = xv_ref[0]
    x = jnp.concatenate([xa, xv], axis=0)
    w = w_ref[...]
    asrc = asrc_ref[...].reshape(1, -1)
    adst = adst_ref[...].reshape(1, -1)

    h = jnp.dot(x, w, preferred_element_type=jnp.float32)

    s_col = jnp.sum(h * asrc, axis=1, keepdims=True)
    t_row = jax.lax.dot_general(adst, h, (((1,), (1,)), ((), ())),
                                preferred_element_type=jnp.float32)

    maxt = jnp.max(t_row, axis=1, keepdims=True)
    y = s_col + maxt
    m = jnp.maximum(y, 0.2 * y)
    a_c = jnp.exp(y - m)
    c_c = jnp.exp(0.2 * y - m)
    b_r = jnp.exp(t_row - maxt)
    d_r = jnp.exp(0.2 * (t_row - maxt))

    page = prob_ref[0]
    C = page.shape[1] // 2
    nodes = jnp.concatenate([page[:, :C], page[:, C:]], axis=0)
    overlap = jax.lax.dot_general(nodes, nodes,
                                  (((1,), (1,)), ((), ())),
                                  preferred_element_type=jnp.float32)
    p = jnp.minimum(jnp.maximum(a_c * b_r, c_c * d_r), overlap)
    rs = jnp.sum(p, axis=1, keepdims=True)

    nact = jnp.sum(nodes.astype(jnp.float32), axis=1, keepdims=True)
    empty = jnp.where(nact > 0.0, 0.0, 1.0)

    rcp = 1.0 / jnp.where(rs > 0.0, rs, 1.0)
    agg = jnp.dot(p, h, preferred_element_type=jnp.float32)
    out = agg * rcp + x + empty * h
    out = jnp.where(out > 0, out, jnp.exp(jnp.minimum(out, 0.0)) - 1.0)

    T = xa.shape[0]
    xa2_ref[0] = out[:T]
    xv2_ref[0] = out[T:]
    f = jax.nn.sigmoid(
        jax.lax.dot_general(out, wprobt_ref[...], (((1,), (1,)), ((), ())),
                            preferred_element_type=jnp.float32)
        + bprob_ref[...].reshape(1, -1))
    fa = f[:T]
    fv = f[T:]
    fp_ref[0, 0] = fa
    fp_ref[0, 1] = fv

    l = jax.lax.dot_general(out, wattt_ref[...], (((1,), (1,)), ((), ())),
                            preferred_element_type=jnp.float32) \
        + batt_ref[...].reshape(1, -1)
    la = l[:T]
    lv = l[T:]
    ea = jnp.exp(la - jnp.max(la, axis=0, keepdims=True))
    ev = jnp.exp(lv - jnp.max(lv, axis=0, keepdims=True))
    num_a = jnp.sum(ea * fa, axis=0, keepdims=True)
    num_v = jnp.sum(ev * fv, axis=0, keepdims=True)
    den_a = jnp.sum(ea, axis=0, keepdims=True)
    den_v = jnp.sum(ev, axis=0, keepdims=True)
    ap_ref[0] = jnp.clip(num_a / den_a, 0.0, 1.0)
    vp_ref[0] = jnp.clip(num_v / den_v, 0.0, 1.0)


def kernel(x_a, x_v, s1_frame_prob, W, a_src, a_dst, W_prob, b_prob, W_att, b_att):
    bs, T, d = x_a.shape
    C = s1_frame_prob.shape[-1]
    n2 = 2 * T

    prob2 = (s1_frame_prob > 0.5).astype(jnp.bfloat16).reshape(bs, T, 2 * C)
    wprobt = W_prob.T
    wattt = W_att.T

    grid = (bs,)
    xa2, xv2, fp, ap, vp = pl.pallas_call(
        _fused_kernel,
        grid=grid,
        in_specs=[
            pl.BlockSpec((1, T, d), lambda b: (b, 0, 0)),
            pl.BlockSpec((1, T, d), lambda b: (b, 0, 0)),
            pl.BlockSpec((1, T, 2 * C), lambda b: (b, 0, 0)),
            pl.BlockSpec((d, d), lambda b: (0, 0)),
            pl.BlockSpec((d,), lambda b: (0,)),
            pl.BlockSpec((d,), lambda b: (0,)),
            pl.BlockSpec((C, d), lambda b: (0, 0)),
            pl.BlockSpec((C,), lambda b: (0,)),
            pl.BlockSpec((C, d), lambda b: (0, 0)),
            pl.BlockSpec((C,), lambda b: (0,)),
        ],
        out_specs=[
            pl.BlockSpec((1, T, d), lambda b: (b, 0, 0)),
            pl.BlockSpec((1, T, d), lambda b: (b, 0, 0)),
            pl.BlockSpec((1, 2, T, C), lambda b: (b, 0, 0, 0)),
            pl.BlockSpec((1, 1, C), lambda b: (b, 0, 0)),
            pl.BlockSpec((1, 1, C), lambda b: (b, 0, 0)),
        ],
        out_shape=[
            jax.ShapeDtypeStruct((bs, T, d), jnp.float32),
            jax.ShapeDtypeStruct((bs, T, d), jnp.float32),
            jax.ShapeDtypeStruct((bs, 2, T, C), jnp.float32),
            jax.ShapeDtypeStruct((bs, 1, C), jnp.float32),
            jax.ShapeDtypeStruct((bs, 1, C), jnp.float32),
        ],
    )(x_a, x_v, prob2, W, a_src, a_dst, wprobt, b_prob, wattt, b_att)

    frame_prob = jnp.transpose(fp, (0, 2, 1, 3))
    a_prob = ap[:, 0, :]
    v_prob = vp[:, 0, :]
    zeros_event = jnp.zeros((bs, C, d), dtype=jnp.float32)
    return (a_prob, v_prob, frame_prob, xa2, xv2, zeros_event, zeros_event)
